# Initial kernel scaffold; baseline (speedup 1.0000x reference)
import jax
import jax.numpy as jnp
from jax.experimental import pallas as pl


def kernel(n_feat, edge_index, W1, b1, W2, b2, W3, b3):
    raise NotImplementedError("write your pallas kernel here")


# speedup vs baseline: 5.0053x; 5.0053x over previous
import jax
import jax.numpy as jnp
from jax import lax
from jax.experimental import pallas as pl
from jax.experimental.pallas import tpu as pltpu
from jax.experimental.pallas import tpu_sc as plsc

N = 10000
E = 320000
D = 128
NC = 2
NS = 16
NW = NC * NS
LPR = 128
ER = E // LPR
ROWS_W = ER // NW
REM = ER - ROWS_W * NW
NP = 10240
NT = NP // NS

_mesh = plsc.VectorSubcoreMesh(
    core_axis_name="c", subcore_axis_name="s", num_cores=NC, num_subcores=NS
)


def _leaky(x):
    return jnp.where(x >= 0, x, 0.01 * x)


def _sc_degrees(src2d, dst2d, zn, cnt_out, idx_v, ones_v, csrc, cdst):
    c = lax.axis_index("c")
    s = lax.axis_index("s")
    wid = s * NC + c

    def fill(j, carry):
        ones_v[pl.ds(j * 16, 16)] = jnp.ones((16,), jnp.float32)
        return carry

    lax.fori_loop(0, LPR // 16, fill, 0)
    sl = pl.ds(s * NT, NT)
    pltpu.sync_copy(zn.at[sl], csrc.at[sl])
    pltpu.sync_copy(zn.at[sl], cdst.at[sl])
    plsc.subcore_barrier()

    def do_row(row):
        pltpu.sync_copy(src2d.at[row], idx_v)
        pltpu.sync_copy(ones_v, csrc.at[idx_v], add=True)
        pltpu.sync_copy(dst2d.at[row], idx_v)
        pltpu.sync_copy(ones_v, cdst.at[idx_v], add=True)

    lax.fori_loop(0, ROWS_W, lambda i, cr: (do_row(wid + i * NW), cr)[1], 0)

    @pl.when(wid < REM)
    def _():
        do_row(wid + ROWS_W * NW)

    plsc.subcore_barrier()
    pltpu.sync_copy(csrc.at[sl], cnt_out.at[0, c, sl])
    pltpu.sync_copy(cdst.at[sl], cnt_out.at[1, c, sl])


_deg_call = pl.kernel(
    _sc_degrees,
    out_type=jax.ShapeDtypeStruct((2, NC, NP), jnp.float32),
    mesh=_mesh,
    scratch_types=[
        pltpu.VMEM((LPR,), jnp.int32),
        pltpu.VMEM((LPR,), jnp.float32),
        pltpu.VMEM_SHARED((NP,), jnp.float32),
        pltpu.VMEM_SHARED((NP,), jnp.float32),
    ],
)


def _sc_agg(z_hbm, src2d, dst2d, zrows, agg_out, idx_s, idx_d, rows_v, shared,
            sem):
    c = lax.axis_index("c")
    s = lax.axis_index("s")
    wid = s * NC + c
    sl = pl.ds(s * NT, NT)
    pltpu.sync_copy(zrows.at[sl], shared.at[sl])
    plsc.subcore_barrier()

    def do_row(row):
        pltpu.sync_copy(src2d.at[row], idx_s)
        pltpu.sync_copy(dst2d.at[row], idx_d)
        pltpu.async_copy(z_hbm.at[idx_s], rows_v, sem).wait()
        pltpu.sync_copy(rows_v, shared.at[idx_d], add=True)

    lax.fori_loop(0, ROWS_W, lambda i, cr: (do_row(wid + i * NW), cr)[1], 0)

    @pl.when(wid < REM)
    def _():
        do_row(wid + ROWS_W * NW)

    plsc.subcore_barrier()
    pltpu.sync_copy(shared.at[sl], agg_out.at[c, sl])


_agg_call = pl.kernel(
    _sc_agg,
    out_type=jax.ShapeDtypeStruct((NC, NP, D), jnp.float32),
    mesh=_mesh,
    scratch_types=[
        pltpu.VMEM((LPR,), jnp.int32),
        pltpu.VMEM((LPR,), jnp.int32),
        pltpu.VMEM((LPR, D), jnp.float32),
        pltpu.VMEM_SHARED((NP, D), jnp.float32),
        pltpu.SemaphoreType.DMA,
    ],
)


def _tc_first_body(x_ref, w_ref, cnt_ref, z_ref, ns_ref, nd_ref):
    cnt = cnt_ref[...]
    deg_o = cnt[0, 0, :N] + cnt[0, 1, :N]
    deg_i = cnt[1, 0, :N] + cnt[1, 1, :N]
    ns = lax.rsqrt(jnp.maximum(deg_o, 1.0))[:, None]
    nd = lax.rsqrt(jnp.maximum(deg_i, 1.0))[:, None]
    u = jnp.dot(x_ref[...], w_ref[...], preferred_element_type=jnp.float32)
    z_ref[...] = u * ns
    ns_ref[...] = ns
    nd_ref[...] = nd


_tc_first = pl.pallas_call(
    _tc_first_body,
    out_shape=(
        jax.ShapeDtypeStruct((N, D), jnp.float32),
        jax.ShapeDtypeStruct((N, 1), jnp.float32),
        jax.ShapeDtypeStruct((N, 1), jnp.float32),
    ),
)


def _tc_mid_body(p_ref, nd_ref, b_ref, w_ref, ns_ref, z_ref):
    agg = (p_ref[0, :N, :] + p_ref[1, :N, :]) * nd_ref[...]
    h = _leaky(agg + b_ref[...][None, :])
    z_ref[...] = (
        jnp.dot(h, w_ref[...], preferred_element_type=jnp.float32)
        * ns_ref[...]
    )


_tc_mid = pl.pallas_call(
    _tc_mid_body,
    out_shape=jax.ShapeDtypeStruct((N, D), jnp.float32),
)


def _tc_final_body(p_ref, nd_ref, b_ref, o_ref):
    agg = (p_ref[0, :N, :] + p_ref[1, :N, :]) * nd_ref[...]
    o_ref[...] = _leaky(agg + b_ref[...][None, :])


_tc_final = pl.pallas_call(
    _tc_final_body,
    out_shape=jax.ShapeDtypeStruct((N, D), jnp.float32),
)


def kernel(n_feat, edge_index, W1, b1, W2, b2, W3, b3):
    src2d = edge_index[0].reshape(ER, LPR)
    dst2d = edge_index[1].reshape(ER, LPR)
    zrows = jnp.zeros((NP, D), jnp.float32)
    zn = jnp.zeros((NP,), jnp.float32)

    cnt = _deg_call(src2d, dst2d, zn)
    z, ns, nd = _tc_first(n_feat, W1, cnt)
    p = _agg_call(z, src2d, dst2d, zrows)
    z = _tc_mid(p, nd, b1, W2, ns)
    p = _agg_call(z, src2d, dst2d, zrows)
    z = _tc_mid(p, nd, b2, W3, ns)
    p = _agg_call(z, src2d, dst2d, zrows)
    return _tc_final(p, nd, b3)


# speedup vs baseline: 10.8013x; 2.1580x over previous
import jax
import jax.numpy as jnp
from jax import lax
from jax.experimental import pallas as pl
from jax.experimental.pallas import tpu as pltpu
from jax.experimental.pallas import tpu_sc as plsc

N = 10000
E = 320000
D = 128
NC = 2
NS = 16
NW = NC * NS
LPR = 128
RW = 80
ERP = NW * RW
EP = ERP * LPR
NP = 10240
NT = NP // NS
RB = 632
RBL = N - RB * (NS - 1)

_mesh = plsc.VectorSubcoreMesh(
    core_axis_name="c", subcore_axis_name="s", num_cores=NC, num_subcores=NS
)


def _leaky(x):
    return jnp.where(x >= 0, x, 0.01 * x)


def _sc_degrees(src3d, dst3d, zn, cnt_out, is3, id3, ones_v, csrc, cdst):
    c = lax.axis_index("c")
    s = lax.axis_index("s")
    wid = s * NC + c

    def fill(j, carry):
        ones_v[pl.ds(j * 16, 16)] = jnp.ones((16,), jnp.float32)
        return carry

    lax.fori_loop(0, LPR // 16, fill, 0)
    sl = pl.ds(s * NT, NT)
    pltpu.sync_copy(zn.at[sl], csrc.at[sl])
    pltpu.sync_copy(zn.at[sl], cdst.at[sl])
    plsc.subcore_barrier()

    base = wid * RW
    pltpu.sync_copy(src3d.at[pl.ds(base, RW)], is3)
    pltpu.sync_copy(dst3d.at[pl.ds(base, RW)], id3)

    def step(i, carry):
        pltpu.sync_copy(ones_v, csrc.at[is3.at[i, 0]], add=True)
        pltpu.sync_copy(ones_v, cdst.at[id3.at[i, 0]], add=True)
        return carry

    lax.fori_loop(0, RW, step, 0)
    plsc.subcore_barrier()
    pltpu.sync_copy(csrc.at[sl], cnt_out.at[0, c, sl])
    pltpu.sync_copy(cdst.at[sl], cnt_out.at[1, c, sl])


_deg_call = pl.kernel(
    _sc_degrees,
    out_type=jax.ShapeDtypeStruct((2, NC, NP), jnp.float32),
    mesh=_mesh,
    scratch_types=[
        pltpu.VMEM((RW, 1, LPR), jnp.int32),
        pltpu.VMEM((RW, 1, LPR), jnp.int32),
        pltpu.VMEM((LPR,), jnp.float32),
        pltpu.VMEM_SHARED((NP,), jnp.float32),
        pltpu.VMEM_SHARED((NP,), jnp.float32),
    ],
)


def _sc_agg(z_hbm, src3d, dst3d, zrows, agg_out, is3, dd0, dd1, b0, b1,
            shared, gs0, gs1, ds0, ds1):
    c = lax.axis_index("c")
    s = lax.axis_index("s")
    wid = s * NC + c

    @pl.when(s < NS - 1)
    def _():
        sl = pl.ds(s * RB, RB)
        pltpu.sync_copy(zrows.at[sl], shared.at[sl])

    @pl.when(s == NS - 1)
    def _():
        sl = pl.ds((NS - 1) * RB, RBL)
        pltpu.sync_copy(zrows.at[sl], shared.at[sl])

    plsc.subcore_barrier()

    base = wid * RW
    pltpu.sync_copy(src3d.at[pl.ds(base, RW)], is3)

    def g_start(i, b, sem):
        pltpu.async_copy(z_hbm.at[is3.at[i, 0]], b, sem)

    def g_wait(i, b, sem):
        pltpu.make_async_copy(z_hbm.at[is3.at[i, 0]], b, sem).wait()

    def d_start(i, dd, sem):
        pltpu.async_copy(dst3d.at[base + i, 0], dd, sem)

    def d_wait(i, dd, sem):
        pltpu.make_async_copy(dst3d.at[base + i, 0], dd, sem).wait()

    d_start(0, dd0, ds0)
    g_start(0, b0, gs0)
    d_start(1, dd1, ds1)
    g_start(1, b1, gs1)

    def body(k, carry):
        i0 = 2 * k
        g_wait(i0, b0, gs0)
        d_wait(i0, dd0, ds0)
        pltpu.sync_copy(b0, shared.at[dd0], add=True)

        @pl.when(i0 + 2 < RW)
        def _():
            g_start(i0 + 2, b0, gs0)
            d_start(i0 + 2, dd0, ds0)

        g_wait(i0 + 1, b1, gs1)
        d_wait(i0 + 1, dd1, ds1)
        pltpu.sync_copy(b1, shared.at[dd1], add=True)

        @pl.when(i0 + 3 < RW)
        def _():
            g_start(i0 + 3, b1, gs1)
            d_start(i0 + 3, dd1, ds1)

        return carry

    lax.fori_loop(0, RW // 2, body, 0)

    plsc.subcore_barrier()

    @pl.when(s < NS - 1)
    def _():
        sl = pl.ds(s * RB, RB)
        pltpu.sync_copy(shared.at[sl], agg_out.at[c, sl])

    @pl.when(s == NS - 1)
    def _():
        sl = pl.ds((NS - 1) * RB, RBL)
        pltpu.sync_copy(shared.at[sl], agg_out.at[c, sl])


_agg_call = pl.kernel(
    _sc_agg,
    out_type=jax.ShapeDtypeStruct((NC, N, D), jnp.float32),
    mesh=_mesh,
    scratch_types=[
        pltpu.VMEM((RW, 1, LPR), jnp.int32),
        pltpu.VMEM((LPR,), jnp.int32),
        pltpu.VMEM((LPR,), jnp.int32),
        pltpu.VMEM((LPR, D), jnp.float32),
        pltpu.VMEM((LPR, D), jnp.float32),
        pltpu.VMEM_SHARED((N, D), jnp.float32),
        pltpu.SemaphoreType.DMA,
        pltpu.SemaphoreType.DMA,
        pltpu.SemaphoreType.DMA,
        pltpu.SemaphoreType.DMA,
    ],
)


def _tc_first_body(x_ref, w_ref, cnt_ref, z_ref, ns_ref, nd_ref):
    cnt = cnt_ref[...]
    deg_o = cnt[0, 0, :N] + cnt[0, 1, :N]
    deg_i = cnt[1, 0, :N] + cnt[1, 1, :N]
    ns = lax.rsqrt(jnp.maximum(deg_o, 1.0))[:, None]
    nd = lax.rsqrt(jnp.maximum(deg_i, 1.0))[:, None]
    u = jnp.dot(x_ref[...], w_ref[...], preferred_element_type=jnp.float32)
    z_ref[:N, :] = u * ns
    z_ref[N:, :] = jnp.zeros((NP - N, D), jnp.float32)
    ns_ref[...] = ns
    nd_ref[...] = nd


_tc_first = pl.pallas_call(
    _tc_first_body,
    out_shape=(
        jax.ShapeDtypeStruct((NP, D), jnp.float32),
        jax.ShapeDtypeStruct((N, 1), jnp.float32),
        jax.ShapeDtypeStruct((N, 1), jnp.float32),
    ),
)


def _tc_mid_body(p_ref, nd_ref, b_ref, w_ref, ns_ref, z_ref):
    agg = (p_ref[0] + p_ref[1]) * nd_ref[...]
    h = _leaky(agg + b_ref[...][None, :])
    z_ref[:N, :] = (
        jnp.dot(h, w_ref[...], preferred_element_type=jnp.float32)
        * ns_ref[...]
    )
    z_ref[N:, :] = jnp.zeros((NP - N, D), jnp.float32)


_tc_mid = pl.pallas_call(
    _tc_mid_body,
    out_shape=jax.ShapeDtypeStruct((NP, D), jnp.float32),
)


def _tc_final_body(p_ref, nd_ref, b_ref, o_ref):
    agg = (p_ref[0] + p_ref[1]) * nd_ref[...]
    o_ref[...] = _leaky(agg + b_ref[...][None, :])


_tc_final = pl.pallas_call(
    _tc_final_body,
    out_shape=jax.ShapeDtypeStruct((N, D), jnp.float32),
)


def kernel(n_feat, edge_index, W1, b1, W2, b2, W3, b3):
    npad = EP - E
    src_pad = N + (jnp.arange(npad, dtype=jnp.int32) % (NP - N))
    dst_agg_pad = jnp.arange(npad, dtype=jnp.int32) % N
    src3d = jnp.concatenate([edge_index[0], src_pad]).reshape(ERP, 1, LPR)
    dstd3d = jnp.concatenate([edge_index[1], src_pad]).reshape(ERP, 1, LPR)
    dsta3d = jnp.concatenate([edge_index[1], dst_agg_pad]).reshape(ERP, 1, LPR)
    zrows = jnp.zeros((NP, D), jnp.float32)
    zn = jnp.zeros((NP,), jnp.float32)

    cnt = _deg_call(src3d, dstd3d, zn)
    z, ns, nd = _tc_first(n_feat, W1, cnt)
    p = _agg_call(z, src3d, dsta3d, zrows)
    z = _tc_mid(p, nd, b1, W2, ns)
    p = _agg_call(z, src3d, dsta3d, zrows)
    z = _tc_mid(p, nd, b2, W3, ns)
    p = _agg_call(z, src3d, dsta3d, zrows)
    return _tc_final(p, nd, b3)


# speedup vs baseline: 11.2680x; 1.0432x over previous
import jax
import jax.numpy as jnp
from jax import lax
from jax.experimental import pallas as pl
from jax.experimental.pallas import tpu as pltpu
from jax.experimental.pallas import tpu_sc as plsc

N = 10000
E = 320000
D = 128
NC = 2
NS = 16
NW = NC * NS
LPR = 128
RW = 80
ERP = NW * RW
EP = ERP * LPR
NP = 10240
NT = NP // NS
RB = 632
RBL = N - RB * (NS - 1)

_mesh = plsc.VectorSubcoreMesh(
    core_axis_name="c", subcore_axis_name="s", num_cores=NC, num_subcores=NS
)


def _leaky(x):
    return jnp.where(x >= 0, x, 0.01 * x)


def _sc_degrees(src3d, dst3d, zn, cnt_out, is3, id3, ones_v, csrc, cdst, csem):
    c = lax.axis_index("c")
    s = lax.axis_index("s")
    wid = s * NC + c

    def fill(j, carry):
        ones_v[pl.ds(j * 16, 16)] = jnp.ones((16,), jnp.float32)
        return carry

    lax.fori_loop(0, LPR // 16, fill, 0)
    sl = pl.ds(s * NT, NT)
    pltpu.sync_copy(zn.at[sl], csrc.at[sl])
    pltpu.sync_copy(zn.at[sl], cdst.at[sl])
    plsc.subcore_barrier()

    base = wid * RW
    pltpu.sync_copy(src3d.at[pl.ds(base, RW)], is3)
    pltpu.sync_copy(dst3d.at[pl.ds(base, RW)], id3)

    def chunk(k, carry):
        i0 = 4 * k
        for j in range(4):
            pltpu.async_copy(ones_v, csrc.at[is3.at[i0 + j, 0]], csem,
                             add=True)
            pltpu.async_copy(ones_v, cdst.at[id3.at[i0 + j, 0]], csem,
                             add=True)
        for j in range(4):
            pltpu.make_async_copy(ones_v, csrc.at[is3.at[i0 + j, 0]],
                                  csem).wait()
            pltpu.make_async_copy(ones_v, cdst.at[id3.at[i0 + j, 0]],
                                  csem).wait()
        return carry

    lax.fori_loop(0, RW // 4, chunk, 0)
    plsc.subcore_barrier()
    pltpu.sync_copy(csrc.at[sl], cnt_out.at[0, c, sl])
    pltpu.sync_copy(cdst.at[sl], cnt_out.at[1, c, sl])


_deg_call = pl.kernel(
    _sc_degrees,
    out_type=jax.ShapeDtypeStruct((2, NC, NP), jnp.float32),
    mesh=_mesh,
    scratch_types=[
        pltpu.VMEM((RW, 1, LPR), jnp.int32),
        pltpu.VMEM((RW, 1, LPR), jnp.int32),
        pltpu.VMEM((LPR,), jnp.float32),
        pltpu.VMEM_SHARED((NP,), jnp.float32),
        pltpu.VMEM_SHARED((NP,), jnp.float32),
        pltpu.SemaphoreType.DMA,
    ],
)


def _sc_agg(z_hbm, src3d, dst3d, zrows, agg_out, is3, dd0, dd1, b0, b1,
            shared, gs0, gs1, ds0, ds1, zs):
    c = lax.axis_index("c")
    s = lax.axis_index("s")
    wid = s * NC + c
    base = wid * RW

    @pl.when(s < NS - 1)
    def _():
        sl = pl.ds(s * RB, RB)
        pltpu.async_copy(zrows.at[sl], shared.at[sl], zs)

    @pl.when(s == NS - 1)
    def _():
        sl = pl.ds((NS - 1) * RB, RBL)
        pltpu.async_copy(zrows.at[sl], shared.at[sl], zs)

    pltpu.sync_copy(src3d.at[pl.ds(base, RW)], is3)

    def g_start(i, b, sem):
        pltpu.async_copy(z_hbm.at[is3.at[i, 0]], b, sem)

    def g_wait(i, b, sem):
        pltpu.make_async_copy(z_hbm.at[is3.at[i, 0]], b, sem).wait()

    def d_start(i, dd, sem):
        pltpu.async_copy(dst3d.at[base + i, 0], dd, sem)

    def d_wait(i, dd, sem):
        pltpu.make_async_copy(dst3d.at[base + i, 0], dd, sem).wait()

    d_start(0, dd0, ds0)
    g_start(0, b0, gs0)
    d_start(1, dd1, ds1)
    g_start(1, b1, gs1)

    @pl.when(s < NS - 1)
    def _():
        sl = pl.ds(s * RB, RB)
        pltpu.make_async_copy(zrows.at[sl], shared.at[sl], zs).wait()

    @pl.when(s == NS - 1)
    def _():
        sl = pl.ds((NS - 1) * RB, RBL)
        pltpu.make_async_copy(zrows.at[sl], shared.at[sl], zs).wait()

    plsc.subcore_barrier()

    def body(k, carry):
        i0 = 2 * k
        g_wait(i0, b0, gs0)
        d_wait(i0, dd0, ds0)
        pltpu.sync_copy(b0, shared.at[dd0], add=True)

        @pl.when(i0 + 2 < RW)
        def _():
            g_start(i0 + 2, b0, gs0)
            d_start(i0 + 2, dd0, ds0)

        g_wait(i0 + 1, b1, gs1)
        d_wait(i0 + 1, dd1, ds1)
        pltpu.sync_copy(b1, shared.at[dd1], add=True)

        @pl.when(i0 + 3 < RW)
        def _():
            g_start(i0 + 3, b1, gs1)
            d_start(i0 + 3, dd1, ds1)

        return carry

    lax.fori_loop(0, RW // 2, body, 0)

    plsc.subcore_barrier()

    @pl.when(s < NS - 1)
    def _():
        sl = pl.ds(s * RB, RB)
        pltpu.sync_copy(shared.at[sl], agg_out.at[c, sl])

    @pl.when(s == NS - 1)
    def _():
        sl = pl.ds((NS - 1) * RB, RBL)
        pltpu.sync_copy(shared.at[sl], agg_out.at[c, sl])


_agg_call = pl.kernel(
    _sc_agg,
    out_type=jax.ShapeDtypeStruct((NC, N, D), jnp.float32),
    mesh=_mesh,
    scratch_types=[
        pltpu.VMEM((RW, 1, LPR), jnp.int32),
        pltpu.VMEM((LPR,), jnp.int32),
        pltpu.VMEM((LPR,), jnp.int32),
        pltpu.VMEM((LPR, D), jnp.float32),
        pltpu.VMEM((LPR, D), jnp.float32),
        pltpu.VMEM_SHARED((N, D), jnp.float32),
        pltpu.SemaphoreType.DMA,
        pltpu.SemaphoreType.DMA,
        pltpu.SemaphoreType.DMA,
        pltpu.SemaphoreType.DMA,
        pltpu.SemaphoreType.DMA,
    ],
)


def _tc_first_body(x_ref, w_ref, cnt_ref, z_ref, ns_ref, nd_ref):
    cnt = cnt_ref[...]
    deg_o = cnt[0, 0, :N] + cnt[0, 1, :N]
    deg_i = cnt[1, 0, :N] + cnt[1, 1, :N]
    ns = lax.rsqrt(jnp.maximum(deg_o, 1.0))[:, None]
    nd = lax.rsqrt(jnp.maximum(deg_i, 1.0))[:, None]
    u = jnp.dot(x_ref[...], w_ref[...], preferred_element_type=jnp.float32)
    z_ref[:N, :] = u * ns
    z_ref[N:, :] = jnp.zeros((NP - N, D), jnp.float32)
    ns_ref[...] = ns
    nd_ref[...] = nd


_tc_first = pl.pallas_call(
    _tc_first_body,
    out_shape=(
        jax.ShapeDtypeStruct((NP, D), jnp.float32),
        jax.ShapeDtypeStruct((N, 1), jnp.float32),
        jax.ShapeDtypeStruct((N, 1), jnp.float32),
    ),
)


def _tc_mid_body(p_ref, nd_ref, b_ref, w_ref, ns_ref, z_ref):
    agg = (p_ref[0] + p_ref[1]) * nd_ref[...]
    h = _leaky(agg + b_ref[...][None, :])
    z_ref[:N, :] = (
        jnp.dot(h, w_ref[...], preferred_element_type=jnp.float32)
        * ns_ref[...]
    )
    z_ref[N:, :] = jnp.zeros((NP - N, D), jnp.float32)


_tc_mid = pl.pallas_call(
    _tc_mid_body,
    out_shape=jax.ShapeDtypeStruct((NP, D), jnp.float32),
)


def _tc_final_body(p_ref, nd_ref, b_ref, o_ref):
    agg = (p_ref[0] + p_ref[1]) * nd_ref[...]
    o_ref[...] = _leaky(agg + b_ref[...][None, :])


_tc_final = pl.pallas_call(
    _tc_final_body,
    out_shape=jax.ShapeDtypeStruct((N, D), jnp.float32),
)


def kernel(n_feat, edge_index, W1, b1, W2, b2, W3, b3):
    npad = EP - E
    src_pad = N + (jnp.arange(npad, dtype=jnp.int32) % (NP - N))
    dst_agg_pad = jnp.arange(npad, dtype=jnp.int32) % N
    src3d = jnp.concatenate([edge_index[0], src_pad]).reshape(ERP, 1, LPR)
    dstd3d = jnp.concatenate([edge_index[1], src_pad]).reshape(ERP, 1, LPR)
    dsta3d = jnp.concatenate([edge_index[1], dst_agg_pad]).reshape(ERP, 1, LPR)
    zrows = jnp.zeros((NP, D), jnp.float32)
    zn = jnp.zeros((NP,), jnp.float32)

    cnt = _deg_call(src3d, dstd3d, zn)
    z, ns, nd = _tc_first(n_feat, W1, cnt)
    p = _agg_call(z, src3d, dsta3d, zrows)
    z = _tc_mid(p, nd, b1, W2, ns)
    p = _agg_call(z, src3d, dsta3d, zrows)
    z = _tc_mid(p, nd, b2, W3, ns)
    p = _agg_call(z, src3d, dsta3d, zrows)
    return _tc_final(p, nd, b3)


# speedup vs baseline: 12.4171x; 1.1020x over previous
import jax
import jax.numpy as jnp
from jax import lax
from jax.experimental import pallas as pl
from jax.experimental.pallas import tpu as pltpu
from jax.experimental.pallas import tpu_sc as plsc

N = 10000
E = 320000
D = 128
NC = 2
NS = 16
NW = NC * NS
LPR = 128
RW = 81
ERP = NW * RW
EP = ERP * LPR
NP = 10240
NT = NP // NS
RB = 632
RBL = N - RB * (NS - 1)

_mesh = plsc.VectorSubcoreMesh(
    core_axis_name="c", subcore_axis_name="s", num_cores=NC, num_subcores=NS
)


def _leaky(x):
    return jnp.where(x >= 0, x, 0.01 * x)


def _sc_degrees(src3d, dst3d, zn, cnt_out, is3, id3, ones_v, csrc, cdst, csem):
    c = lax.axis_index("c")
    s = lax.axis_index("s")
    wid = s * NC + c

    def fill(j, carry):
        ones_v[pl.ds(j * 16, 16)] = jnp.ones((16,), jnp.float32)
        return carry

    lax.fori_loop(0, LPR // 16, fill, 0)
    sl = pl.ds(s * NT, NT)
    pltpu.sync_copy(zn.at[sl], csrc.at[sl])
    pltpu.sync_copy(zn.at[sl], cdst.at[sl])
    plsc.subcore_barrier()

    base = wid * RW
    pltpu.sync_copy(src3d.at[pl.ds(base, RW)], is3)
    pltpu.sync_copy(dst3d.at[pl.ds(base, RW)], id3)

    def chunk(k, carry):
        i0 = 4 * k
        for j in range(4):
            pltpu.async_copy(ones_v, csrc.at[is3.at[i0 + j, 0]], csem,
                             add=True)
            pltpu.async_copy(ones_v, cdst.at[id3.at[i0 + j, 0]], csem,
                             add=True)
        for j in range(4):
            pltpu.make_async_copy(ones_v, csrc.at[is3.at[i0 + j, 0]],
                                  csem).wait()
            pltpu.make_async_copy(ones_v, cdst.at[id3.at[i0 + j, 0]],
                                  csem).wait()
        return carry

    lax.fori_loop(0, RW // 4, chunk, 0)
    for i in range(4 * (RW // 4), RW):
        pltpu.sync_copy(ones_v, csrc.at[is3.at[i, 0]], add=True)
        pltpu.sync_copy(ones_v, cdst.at[id3.at[i, 0]], add=True)
    plsc.subcore_barrier()
    pltpu.sync_copy(csrc.at[sl], cnt_out.at[0, c, sl])
    pltpu.sync_copy(cdst.at[sl], cnt_out.at[1, c, sl])


_deg_call = pl.kernel(
    _sc_degrees,
    out_type=jax.ShapeDtypeStruct((2, NC, NP), jnp.float32),
    mesh=_mesh,
    scratch_types=[
        pltpu.VMEM((RW, 1, LPR), jnp.int32),
        pltpu.VMEM((RW, 1, LPR), jnp.int32),
        pltpu.VMEM((LPR,), jnp.float32),
        pltpu.VMEM_SHARED((NP,), jnp.float32),
        pltpu.VMEM_SHARED((NP,), jnp.float32),
        pltpu.SemaphoreType.DMA,
    ],
)


def _sc_agg(z_hbm, src3d, dst3d, zrows, agg_out, ss3, dd3, b0, b1, b2,
            shared, g0, g1, g2, i0s, i1s, i2s, i3s, i4s, i5s, zs):
    c = lax.axis_index("c")
    s = lax.axis_index("s")
    wid = s * NC + c
    base = wid * RW

    @pl.when(s < NS - 1)
    def _():
        sl = pl.ds(s * RB, RB)
        pltpu.async_copy(zrows.at[sl], shared.at[sl], zs)

    @pl.when(s == NS - 1)
    def _():
        sl = pl.ds((NS - 1) * RB, RBL)
        pltpu.async_copy(zrows.at[sl], shared.at[sl], zs)

    bufs = (b0, b1, b2)
    gsems = (g0, g1, g2)
    isems = (i0s, i1s, i2s, i3s, i4s, i5s)

    def i_start(i, slot):
        pltpu.async_copy(src3d.at[base + i, 0], ss3.at[slot, 0], isems[slot])
        pltpu.async_copy(dst3d.at[base + i, 0], dd3.at[slot, 0], isems[slot])

    def i_wait(i, slot):
        pltpu.make_async_copy(src3d.at[base + i, 0], ss3.at[slot, 0],
                              isems[slot]).wait()
        pltpu.make_async_copy(dst3d.at[base + i, 0], dd3.at[slot, 0],
                              isems[slot]).wait()

    def g_start(slot, bj):
        pltpu.async_copy(z_hbm.at[ss3.at[slot, 0]], bufs[bj], gsems[bj])

    def g_wait(bj):
        pltpu.make_async_copy(z_hbm.at[ss3.at[0, 0]], bufs[bj],
                              gsems[bj]).wait()

    for j in range(6):
        i_start(j, j)
    for j in range(3):
        i_wait(j, j)
        g_start(j, j)

    @pl.when(s < NS - 1)
    def _():
        sl = pl.ds(s * RB, RB)
        pltpu.make_async_copy(zrows.at[sl], shared.at[sl], zs).wait()

    @pl.when(s == NS - 1)
    def _():
        sl = pl.ds((NS - 1) * RB, RBL)
        pltpu.make_async_copy(zrows.at[sl], shared.at[sl], zs).wait()

    plsc.subcore_barrier()

    def chunk(k, carry):
        ibase = 6 * k
        for u in range(6):
            i = ibase + u
            bj = u % 3
            g_wait(bj)
            pltpu.sync_copy(bufs[bj], shared.at[dd3.at[u, 0]], add=True)

            @pl.when(i + 3 < RW)
            def _(i=i, u=u, bj=bj):
                i_wait(i + 3, (u + 3) % 6)
                g_start((u + 3) % 6, bj)

            @pl.when(i + 6 < RW)
            def _(i=i, u=u):
                i_start(i + 6, u)

        return carry

    lax.fori_loop(0, RW // 6, chunk, 0)
    for u in range(RW - 6 * (RW // 6)):
        g_wait(u % 3)
        pltpu.sync_copy(bufs[u % 3], shared.at[dd3.at[u, 0]], add=True)

    plsc.subcore_barrier()

    @pl.when(s < NS - 1)
    def _():
        sl = pl.ds(s * RB, RB)
        pltpu.sync_copy(shared.at[sl], agg_out.at[c, sl])

    @pl.when(s == NS - 1)
    def _():
        sl = pl.ds((NS - 1) * RB, RBL)
        pltpu.sync_copy(shared.at[sl], agg_out.at[c, sl])


_agg_call = pl.kernel(
    _sc_agg,
    out_type=jax.ShapeDtypeStruct((NC, N, D), jnp.float32),
    mesh=_mesh,
    scratch_types=[
        pltpu.VMEM((6, 1, LPR), jnp.int32),
        pltpu.VMEM((6, 1, LPR), jnp.int32),
        pltpu.VMEM((LPR, D), jnp.float32),
        pltpu.VMEM((LPR, D), jnp.float32),
        pltpu.VMEM((LPR, D), jnp.float32),
        pltpu.VMEM_SHARED((N, D), jnp.float32),
    ] + [pltpu.SemaphoreType.DMA] * 10,
)


def _tc_first_body(x_ref, w_ref, cnt_ref, z_ref, ns_ref, nd_ref):
    cnt = cnt_ref[...]
    deg_o = cnt[0, 0, :N] + cnt[0, 1, :N]
    deg_i = cnt[1, 0, :N] + cnt[1, 1, :N]
    ns = lax.rsqrt(jnp.maximum(deg_o, 1.0))[:, None]
    nd = lax.rsqrt(jnp.maximum(deg_i, 1.0))[:, None]
    u = jnp.dot(x_ref[...], w_ref[...], preferred_element_type=jnp.float32)
    z_ref[:N, :] = u * ns
    z_ref[N:, :] = jnp.zeros((NP - N, D), jnp.float32)
    ns_ref[...] = ns
    nd_ref[...] = nd


_tc_first = pl.pallas_call(
    _tc_first_body,
    out_shape=(
        jax.ShapeDtypeStruct((NP, D), jnp.float32),
        jax.ShapeDtypeStruct((N, 1), jnp.float32),
        jax.ShapeDtypeStruct((N, 1), jnp.float32),
    ),
)


def _tc_mid_body(p_ref, nd_ref, b_ref, w_ref, ns_ref, z_ref):
    agg = (p_ref[0] + p_ref[1]) * nd_ref[...]
    h = _leaky(agg + b_ref[...][None, :])
    z_ref[:N, :] = (
        jnp.dot(h, w_ref[...], preferred_element_type=jnp.float32)
        * ns_ref[...]
    )
    z_ref[N:, :] = jnp.zeros((NP - N, D), jnp.float32)


_tc_mid = pl.pallas_call(
    _tc_mid_body,
    out_shape=jax.ShapeDtypeStruct((NP, D), jnp.float32),
)


def _tc_final_body(p_ref, nd_ref, b_ref, o_ref):
    agg = (p_ref[0] + p_ref[1]) * nd_ref[...]
    o_ref[...] = _leaky(agg + b_ref[...][None, :])


_tc_final = pl.pallas_call(
    _tc_final_body,
    out_shape=jax.ShapeDtypeStruct((N, D), jnp.float32),
)


def kernel(n_feat, edge_index, W1, b1, W2, b2, W3, b3):
    npad = EP - E
    src_pad = N + (jnp.arange(npad, dtype=jnp.int32) % (NP - N))
    dst_agg_pad = jnp.arange(npad, dtype=jnp.int32) % N
    src3d = jnp.concatenate([edge_index[0], src_pad]).reshape(ERP, 1, LPR)
    dstd3d = jnp.concatenate([edge_index[1], src_pad]).reshape(ERP, 1, LPR)
    dsta3d = jnp.concatenate([edge_index[1], dst_agg_pad]).reshape(ERP, 1, LPR)
    zrows = jnp.zeros((NP, D), jnp.float32)
    zn = jnp.zeros((NP,), jnp.float32)

    cnt = _deg_call(src3d, dstd3d, zn)
    z, ns, nd = _tc_first(n_feat, W1, cnt)
    p = _agg_call(z, src3d, dsta3d, zrows)
    z = _tc_mid(p, nd, b1, W2, ns)
    p = _agg_call(z, src3d, dsta3d, zrows)
    z = _tc_mid(p, nd, b2, W3, ns)
    p = _agg_call(z, src3d, dsta3d, zrows)
    return _tc_final(p, nd, b3)
